# baseline (speedup 1.0000x reference)
import functools
import math

import jax
import jax.numpy as jnp
from jax import lax
from jax.experimental import pallas as pl
from jax.experimental.pallas import tpu as pltpu
from jax.experimental.pallas import tpu_sc as plsc

N_EMB = 64
SCALE = math.sqrt(N_EMB)
LANES = 16

NC = 2
NS = 16
NW = NC * NS

CHUNK = 128


def _emb_body(b_per_w, n_chunks, x_hbm, lut_hbm, out_hbm, idx_v, rows_v, sem):
    wid = lax.axis_index("s") * NC + lax.axis_index("c")
    base = wid * b_per_w
    pltpu.sync_copy(x_hbm.at[pl.ds(base, b_per_w)], idx_v)

    def chunk_body(j, carry):
        idx_c = idx_v.at[pl.ds(j * CHUNK, CHUNK)]
        pltpu.async_copy(lut_hbm.at[idx_c], rows_v, sem).wait()

        def row_body(i, carry2):
            for v in range(N_EMB // LANES):
                sl = pl.ds(v * LANES, LANES)
                rows_v[i, sl] = rows_v[i, sl] * SCALE
            return carry2

        lax.fori_loop(0, CHUNK, row_body, 0, unroll=2)

        pltpu.sync_copy(rows_v, out_hbm.at[pl.ds(base + j * CHUNK, CHUNK)])
        return carry

    lax.fori_loop(0, n_chunks, chunk_body, 0)


def kernel(x, lut):
    B = x.shape[0] * x.shape[1]
    b_per_w = B // NW
    n_chunks = b_per_w // CHUNK
    x_flat = x.reshape(B).astype(jnp.int32)

    mesh = plsc.VectorSubcoreMesh(core_axis_name="c", subcore_axis_name="s")
    run = functools.partial(
        pl.kernel,
        out_type=jax.ShapeDtypeStruct((B, N_EMB), jnp.float32),
        mesh=mesh,
        scratch_types=[
            pltpu.VMEM((b_per_w,), jnp.int32),
            pltpu.VMEM((CHUNK, N_EMB), jnp.float32),
            pltpu.SemaphoreType.DMA,
        ],
        compiler_params=pltpu.CompilerParams(use_tc_tiling_on_sc=False),
    )(functools.partial(_emb_body, b_per_w, n_chunks))

    out = run(x_flat, lut)
    return out.reshape(x.shape[0], x.shape[1], N_EMB)
